# baseline (speedup 1.0000x reference)
import functools

import jax
import jax.numpy as jnp
from jax.experimental import pallas as pl
from jax.experimental.pallas import tpu as pltpu

CLIP_LENGTH = 256.0
OVERLAP_THRESH = 0.5
EPS = float(jnp.finfo(jnp.float32).eps)
SMOOTH = 1e-4
MAXN = CLIP_LENGTH * 2.0

PB = 2048
PM = PB // 8


def _loss_body(ngt, c_sz, tgt_ref, loc_ref, conf_ref, ploc_ref, pconf_ref,
               center_ref, priors_ref, out_ref):
    b = pl.program_id(0)
    i = pl.program_id(1)

    @pl.when((b == 0) & (i == 0))
    def _init():
        for k in range(8):
            out_ref[k] = 0.0

    pc = priors_ref[0]
    ll = loc_ref[0, 0, 0]
    lr = loc_ref[0, 0, 1]

    best_area = jnp.full((8, PM), jnp.inf, jnp.float32)
    bt0 = jnp.zeros((8, PM), jnp.float32)
    bt1 = jnp.zeros((8, PM), jnp.float32)
    blab = jnp.zeros((8, PM), jnp.float32)
    for j in range(ngt):
        t0 = tgt_ref[b, j, 0]
        t1 = tgt_ref[b, j, 1]
        lab = tgt_ref[b, j, 2]
        left = (pc - t0) * CLIP_LENGTH
        right = (t1 - pc) * CLIP_LENGTH
        area = left + right
        area = jnp.where((left < 0.0) | (right < 0.0), MAXN, area)
        take = area < best_area
        best_area = jnp.where(take, area, best_area)
        bt0 = jnp.where(take, t0, bt0)
        bt1 = jnp.where(take, t1, bt1)
        blab = jnp.where(take, lab, blab)

    lt_l = (pc - bt0) * CLIP_LENGTH
    lt_r = (bt1 - pc) * CLIP_LENGTH
    conf_t = jnp.where(best_area >= MAXN, 0.0, blab)

    inter = jnp.minimum(ll, lt_l) + jnp.minimum(lr, lt_r)
    union = (lt_l + lt_r) + (ll + lr) - inter
    iou = inter / jnp.maximum(union, EPS)
    prop_conf_t = jnp.where(iou < OVERLAP_THRESH, 0.0, conf_t)

    posf = (conf_t > 0.0).astype(jnp.float32)
    ppf = (prop_conf_t > 0.0).astype(jnp.float32)

    ac = jnp.maximum(ll, lt_l) + jnp.maximum(lr, lt_r)
    giou = iou - (ac - union) / jnp.maximum(ac, EPS)
    loss_l = jnp.sum((1.0 - giou) * posf)

    prop_w = ll + lr
    inv_hw = 1.0 / (0.5 * prop_w)
    plt_l = (lt_l - ll) * inv_hw
    plt_r = (lt_r - lr) * inv_hw
    pll = ploc_ref[0, 0, 0]
    plr = ploc_ref[0, 0, 1]
    loss_prop_l = jnp.sum((jnp.abs(pll - plt_l) + jnp.abs(plr - plt_r)) * ppf)

    cl = 0.5 * prop_w * pll + ll
    cr = 0.5 * prop_w * plr + lr
    inter2 = jnp.minimum(cl, lt_l) + jnp.minimum(cr, lt_r)
    union2 = (lt_l + lt_r) + (cl + cr) - inter2
    iou2 = jnp.maximum(inter2 / jnp.maximum(union2, EPS), 0.0)
    x = center_ref[0, 0]
    bce = jnp.maximum(x, 0.0) - x * iou2 + jnp.log1p(jnp.exp(-jnp.abs(x)))
    loss_ct = jnp.sum(bce * posf)

    def focal(z, lab_i):
        z3 = z.T.reshape(8, c_sz, PM)
        m = jnp.max(z3, axis=1, keepdims=True)
        e = jnp.exp(z3 - m)
        s = jnp.sum(e, axis=1)
        cls = jax.lax.broadcasted_iota(jnp.int32, (8, c_sz, PM), 1)
        et = jnp.sum(jnp.where(cls == lab_i[:, None, :], e, 0.0), axis=1)
        pt = jnp.clip(et / s, SMOOTH, 1.0 - SMOOTH)
        at = jnp.where(lab_i == 0, 0.25, 0.75)
        return jnp.sum(-at * (1.0 - pt) * (1.0 - pt) * jnp.log(pt))

    lab_conf = conf_t.astype(jnp.int32)
    lab_prop = prop_conf_t.astype(jnp.int32)
    loss_c = focal(conf_ref[0], lab_conf)
    loss_prop_c = focal(pconf_ref[0], lab_prop)

    out_ref[0] += loss_l
    out_ref[1] += loss_c
    out_ref[2] += loss_prop_l
    out_ref[3] += loss_prop_c
    out_ref[4] += loss_ct
    out_ref[5] += jnp.sum(posf)
    out_ref[6] += jnp.sum(ppf)


def _to_tiles(arr, b_sz, nblk):
    return arr.reshape(b_sz, nblk, PM, 8).transpose(0, 1, 3, 2)


@jax.jit
def kernel(loc_data, conf_data, prop_loc_data, prop_conf_data, center_data,
           priors, act_data, prop_act_data, targets):
    b_sz, p_sz, c_sz = conf_data.shape
    ngt = targets.shape[1]
    nblk = p_sz // PB

    locR = jnp.stack([_to_tiles(loc_data[..., 0], b_sz, nblk),
                      _to_tiles(loc_data[..., 1], b_sz, nblk)], axis=2)
    plocR = jnp.stack([_to_tiles(prop_loc_data[..., 0], b_sz, nblk),
                       _to_tiles(prop_loc_data[..., 1], b_sz, nblk)], axis=2)
    centerR = _to_tiles(center_data[..., 0], b_sz, nblk)
    priorsR = _to_tiles(priors[:, 0][None], 1, nblk)[0]
    confR = conf_data.reshape(b_sz, p_sz // 8, 8 * c_sz)
    pconfR = prop_conf_data.reshape(b_sz, p_sz // 8, 8 * c_sz)

    sums = pl.pallas_call(
        functools.partial(_loss_body, ngt, c_sz),
        grid=(b_sz, nblk),
        in_specs=[
            pl.BlockSpec(memory_space=pltpu.SMEM),
            pl.BlockSpec((1, 1, 2, 8, PM), lambda b, i: (b, i, 0, 0, 0)),
            pl.BlockSpec((1, PM, 8 * c_sz), lambda b, i: (b, i, 0)),
            pl.BlockSpec((1, 1, 2, 8, PM), lambda b, i: (b, i, 0, 0, 0)),
            pl.BlockSpec((1, PM, 8 * c_sz), lambda b, i: (b, i, 0)),
            pl.BlockSpec((1, 1, 8, PM), lambda b, i: (b, i, 0, 0)),
            pl.BlockSpec((1, 8, PM), lambda b, i: (i, 0, 0)),
        ],
        out_specs=pl.BlockSpec(memory_space=pltpu.SMEM),
        out_shape=jax.ShapeDtypeStruct((8,), jnp.float32),
    )(targets, locR, confR, plocR, pconfR, centerR, priorsR)

    n = jnp.maximum(sums[5], 1.0)
    pn = jnp.maximum(sums[6], 1.0)
    return jnp.stack([sums[0] / n, sums[1] / n, sums[2] / pn,
                      sums[3] / pn, sums[4] / n])
